# Initial kernel scaffold; baseline (speedup 1.0000x reference)
import jax
import jax.numpy as jnp
from jax.experimental import pallas as pl


def kernel(x, indices, values_ih, values_hh, bias, W1, b1, W2, b2, num_steps):
    raise NotImplementedError("write your pallas kernel here")


# speedup vs baseline: 40.8639x; 40.8639x over previous
import functools

import jax
import jax.numpy as jnp
from jax import lax
from jax.experimental import pallas as pl
from jax.experimental.pallas import tpu as pltpu
from jax.experimental.pallas import tpu_sc as plsc

N = 45000
B = 16
SP = 33
NC, NS = 2, 16
NW = NC * NS
CPT = 1408
NPAD = CPT * NW
EPT = CPT * SP
NCHUNK = 64
ECH = NCHUNK * SP
NCHUNKS = CPT // NCHUNK
GS = 64
SCCH = ECH // GS
ZROWS = NPAD // NS
ZCP = 704
NNZPAD = NPAD * SP

_MESH = plsc.VectorSubcoreMesh(core_axis_name="c", subcore_axis_name="s")


def _make_spmm(with_prologue: bool):
    n_in = 3 if with_prologue else 1

    scratch = [
        pltpu.VMEM_SHARED((NPAD, B), jnp.float32),
        pltpu.VMEM((NCHUNK, B), jnp.float32),
        pltpu.VMEM((NCHUNK, B), jnp.float32),
        pltpu.VMEM((NCHUNK, B), jnp.float32),
        pltpu.VMEM((NCHUNK, B), jnp.float32),
        pltpu.VMEM((ECH, B), jnp.float32),
        pltpu.VMEM((ECH,), jnp.float32),
        pltpu.VMEM((SCCH, GS), jnp.int32),
    ]

    @functools.partial(
        pl.kernel,
        out_type=jax.ShapeDtypeStruct((NC, NPAD, B), jnp.float32),
        mesh=_MESH,
        scratch_types=scratch,
        compiler_params=pltpu.CompilerParams(use_tc_tiling_on_sc=False),
    )
    def spmm(*refs):
        ins = refs[:n_in + 2]
        out = refs[n_in + 2]
        accum, hbuf, d0, d1, d2, contrib, vbuf, rbuf = refs[n_in + 3:]
        vals = ins[n_in]
        rows3d = ins[n_in + 1]

        cid = lax.axis_index("c")
        sid = lax.axis_index("s")
        wid = cid * NS + sid

        def zrow(i, c):
            contrib[i, :] = jnp.zeros((B,), jnp.float32)
            return c
        lax.fori_loop(0, ZCP, zrow, 0)
        zbase = sid * ZROWS
        for i in range(ZROWS // ZCP):
            pltpu.sync_copy(contrib.at[pl.ds(0, ZCP)],
                            accum.at[pl.ds(zbase + i * ZCP, ZCP)])
        plsc.subcore_barrier()

        nbase0 = wid * CPT
        ebase0 = wid * EPT

        def chunk(ch, c):
            nbase = nbase0 + ch * NCHUNK
            ebase = ebase0 + ch * ECH
            cidx = wid * NCHUNKS + ch
            if with_prologue:
                pltpu.sync_copy(ins[0].at[pl.ds(nbase, NCHUNK)], d0)
                pltpu.sync_copy(ins[1].at[pl.ds(nbase, NCHUNK)], d1)
                pltpu.sync_copy(ins[2].at[pl.ds(nbase, NCHUNK)], d2)
            else:
                pltpu.sync_copy(ins[0].at[pl.ds(nbase, NCHUNK)], hbuf)
            pltpu.sync_copy(vals.at[pl.ds(ebase, ECH)], vbuf)
            pltpu.sync_copy(rows3d.at[cidx], rbuf)

            if with_prologue:
                def hrow(n, cc):
                    hbuf[n, :] = jnp.maximum(d0[n, :] + d1[n, :] + d2[n, :], 0.0)
                    return cc
                lax.fori_loop(0, NCHUNK, hrow, 0)

            def erow(n, cc):
                hv = hbuf[n, :]
                e0 = n * SP
                va = vbuf[pl.ds(e0, 16)]
                vb = vbuf[pl.ds(e0 + 16, 16)]
                vc = vbuf[pl.ds(e0 + 17, 16)]
                for s in range(16):
                    contrib[e0 + s, :] = hv * va[s]
                for s in range(16):
                    contrib[e0 + 16 + s, :] = hv * vb[s]
                contrib[e0 + 32, :] = hv * vc[15]
                return cc
            lax.fori_loop(0, NCHUNK, erow, 0)

            for j in range(SCCH):
                pltpu.sync_copy(contrib.at[pl.ds(j * GS, GS)],
                                accum.at[rbuf.at[j]], add=True)
            return c

        lax.fori_loop(0, NCHUNKS, chunk, 0)

        plsc.subcore_barrier()
        pltpu.sync_copy(accum.at[pl.ds(zbase, ZROWS)],
                        out.at[cid, pl.ds(zbase, ZROWS)])

    return spmm


_spmm_ih = _make_spmm(with_prologue=False)
_spmm_hh = _make_spmm(with_prologue=True)

_HBLK = 4096


def _combine_body(a_ref, b_ref, c_ref, o_ref):
    o_ref[...] = a_ref[...] + b_ref[...] + c_ref[...]


def _combine(inp_a, inp_b, bias2d):
    return pl.pallas_call(
        _combine_body,
        grid=(NPAD // _HBLK,),
        in_specs=[
            pl.BlockSpec((_HBLK, B), lambda k: (k, 0)),
            pl.BlockSpec((_HBLK, B), lambda k: (k, 0)),
            pl.BlockSpec((_HBLK, 1), lambda k: (k, 0)),
        ],
        out_specs=pl.BlockSpec((_HBLK, B), lambda k: (k, 0)),
        out_shape=jax.ShapeDtypeStruct((NPAD, B), jnp.float32),
    )(inp_a, inp_b, bias2d)


def _head_body(d_ref, a_ref, b_ref, w1_ref, b1_ref, w2_ref, b2_ref,
               o_ref, acc_ref):
    k = pl.program_id(0)

    @pl.when(k == 0)
    def _():
        acc_ref[...] = jnp.zeros_like(acc_ref)

    h = jnp.maximum(d_ref[...] + a_ref[...] + b_ref[...], 0.0)
    acc_ref[...] += lax.dot_general(
        h, w1_ref[...], (((0,), (1,)), ((), ())),
        preferred_element_type=jnp.float32)

    @pl.when(k == pl.num_programs(0) - 1)
    def _():
        o1 = jnp.maximum(acc_ref[...] + b1_ref[...], 0.0)
        o_ref[...] = lax.dot_general(
            o1, w2_ref[...], (((1,), (1,)), ((), ())),
            preferred_element_type=jnp.float32) + b2_ref[...]


def _head(drive, s_a, s_b, w1p, b1_2d, w2, b2_2d, nclass):
    return pl.pallas_call(
        _head_body,
        grid=(NPAD // _HBLK,),
        in_specs=[
            pl.BlockSpec((_HBLK, B), lambda k: (k, 0)),
            pl.BlockSpec((_HBLK, B), lambda k: (k, 0)),
            pl.BlockSpec((_HBLK, B), lambda k: (k, 0)),
            pl.BlockSpec((64, _HBLK), lambda k: (0, k)),
            pl.BlockSpec((1, 64), lambda k: (0, 0)),
            pl.BlockSpec((nclass, 64), lambda k: (0, 0)),
            pl.BlockSpec((1, nclass), lambda k: (0, 0)),
        ],
        out_specs=pl.BlockSpec((B, nclass), lambda k: (0, 0)),
        out_shape=jax.ShapeDtypeStruct((B, nclass), jnp.float32),
        scratch_shapes=[pltpu.VMEM((B, 64), jnp.float32)],
    )(drive, s_a, s_b, w1p, b1_2d, w2, b2_2d)


def kernel(x, indices, values_ih, values_hh, bias, W1, b1, W2, b2, num_steps):
    nnz = indices.shape[1]
    pad_e = NNZPAD - nnz
    rows = indices[0].astype(jnp.int32)
    rows3d = jnp.pad(rows, (0, pad_e)).reshape(-1, SCCH, GS)
    vih = jnp.pad(values_ih.astype(jnp.float32), (0, pad_e))
    vhh = jnp.pad(values_hh.astype(jnp.float32), (0, pad_e))
    x_t = jnp.pad(x.T, ((0, NPAD - N), (0, 0)))
    bias2d = jnp.pad(bias, (0, NPAD - N))[:, None]
    w1p = jnp.pad(W1, ((0, 0), (0, NPAD - N)))
    nclass = W2.shape[0]

    inp = _spmm_ih(x_t, vih, rows3d)
    drive = _combine(inp[0], inp[1], bias2d)

    scat0 = jnp.zeros((NC, NPAD, B), jnp.float32)

    def step(_, scat):
        return _spmm_hh(drive, scat[0], scat[1], vhh, rows3d)

    scat = lax.fori_loop(0, num_steps - 1, step, scat0)
    return _head(drive, scat[0], scat[1], w1p, b1[None, :], W2, b2[None, :],
                 nclass)
